# baseline (speedup 1.0000x reference)
import functools
import math

import jax
import jax.numpy as jnp
from jax.experimental import pallas as pl
from jax.experimental.pallas import tpu as pltpu
from jax.experimental.pallas import tpu_sc as plsc

NN = 10000
BG = 100
WIN = 256
HD = 128
KTOP = 16
NE = 320000
NPAD = 10496
NT = NPAD // WIN
NWORK = 32
ECHUNK = 128
NCH = 80
GDEPTH = 2
STRIPE = NPAD // 16


def _enc_body(x_ref, w_ref, b_ref, o_ref):
    o_ref[...] = jnp.dot(x_ref[...], w_ref[...],
                         preferred_element_type=jnp.float32) + b_ref[...]


def _ffn_body(h_ref, w1_ref, b1_ref, w2_ref, b2_ref, o_ref, st_ref):
    t = pl.program_id(0)
    h = h_ref[...]
    ff = jnp.maximum(jnp.dot(h, w1_ref[...], preferred_element_type=jnp.float32)
                     + b1_ref[...], 0.0)
    o = h + (jnp.dot(ff, w2_ref[...], preferred_element_type=jnp.float32)
             + b2_ref[...])
    o_ref[...] = o
    rows = t * WIN + jax.lax.broadcasted_iota(jnp.int32, (WIN, 1), 0)
    om = jnp.where(rows < NN, o, 0.0)
    s1 = jnp.sum(om, axis=0)
    s2 = jnp.sum(om * om, axis=0)
    blk = jnp.concatenate([s1[None, :], s2[None, :],
                           jnp.zeros((6, HD), jnp.float32)], axis=0)

    @pl.when(t == 0)
    def _():
        st_ref[...] = blk

    @pl.when(t > 0)
    def _():
        st_ref[...] = st_ref[...] + blk


def _bn_body(h_ref, st_ref, sc_ref, bi_ref, o_ref):
    mu = st_ref[0, :] / float(NN)
    var = st_ref[1, :] / float(NN) - mu * mu
    inv = jax.lax.rsqrt(var + 1e-5)
    o_ref[...] = sc_ref[...] * (h_ref[...] - mu) * inv + bi_ref[...]


def _attn_body(offs_ref, cnts_ref, h_ref, wq_ref, wk_ref, o_ref):
    b = pl.program_id(0)
    off = offs_ref[b]
    cnt = cnts_ref[b]
    win = h_ref[pl.ds(off, WIN), :]
    rows = jax.lax.broadcasted_iota(jnp.int32, (WIN, 1), 0)
    winz = jnp.where(rows < cnt, win, 0.0)
    q0 = jnp.dot(winz, wq_ref[0], preferred_element_type=jnp.float32)
    k0 = jnp.dot(winz, wk_ref[0], preferred_element_type=jnp.float32)
    q1 = jnp.dot(winz, wq_ref[1], preferred_element_type=jnp.float32)
    k1 = jnp.dot(winz, wk_ref[1], preferred_element_type=jnp.float32)
    dn = (([1], [1]), ((), ()))
    sq = jnp.float32(math.sqrt(float(HD)))
    s0 = jax.lax.dot_general(q0, k0, dn, preferred_element_type=jnp.float32)
    s1 = jax.lax.dot_general(q1, k1, dn, preferred_element_type=jnp.float32)
    s = (s0 / sq + s1 / sq) * 0.5
    cols = jax.lax.broadcasted_iota(jnp.int32, (WIN, WIN), 1)
    sm = jnp.where(cols < cnt, s, -1e9)

    def body(_, sw):
        m = jnp.max(sw, axis=1, keepdims=True)
        return jnp.where(sw == m, -jnp.inf, sw)

    sw = jax.lax.fori_loop(0, KTOP - 1, body, sm)
    thr = jnp.max(sw, axis=1, keepdims=True)
    wmat = (sm >= thr).astype(jnp.float32)
    o_ref[0] = jnp.dot(wmat, winz, preferred_element_type=jnp.float32,
                       precision=jax.lax.Precision.HIGHEST)


def _gin_body(offs_ref, cnts_ref, h_ref, p_ref, eps_ref, agg_ref, w1_ref,
              b1_ref, w2_ref, b2_ref, o_ref, pool_ref, *, with_pool):
    b = pl.program_id(0)
    off = offs_ref[b]
    cnt = cnts_ref[b]

    @pl.when(b == 0)
    def _():
        o_ref[...] = jnp.zeros((NPAD, HD), jnp.float32)

    hw_in = h_ref[pl.ds(off, WIN), :]
    g = (((1.0 + eps_ref[0, 0]) * hw_in + agg_ref[0])
         + (p_ref[0, pl.ds(off, WIN), :] + p_ref[1, pl.ds(off, WIN), :]))
    hh = jnp.maximum(jnp.dot(g, w1_ref[...], preferred_element_type=jnp.float32)
                     + b1_ref[...], 0.0)
    hw = jnp.maximum(jnp.dot(hh, w2_ref[...], preferred_element_type=jnp.float32)
                     + b2_ref[...], 0.0)
    o_ref[pl.ds(off, WIN), :] = hw
    if with_pool:
        rows = jax.lax.broadcasted_iota(jnp.int32, (WIN, 1), 0)
        hm = jnp.where(rows < cnt, hw, 0.0)
        srow = jnp.sum(hm, axis=0) / jnp.maximum(cnt, 1).astype(jnp.float32)
        pool_ref[0] = jnp.broadcast_to(srow[None, :], (8, HD))


def _head_body(p_ref, w1_ref, b1_ref, w2_ref, b2_ref, o_ref):
    p = p_ref[:, 0, :]
    hh = jnp.maximum(jnp.dot(p, w1_ref[...], preferred_element_type=jnp.float32)
                     + b1_ref[...], 0.0)
    o_ref[...] = jnp.dot(hh, w2_ref[...], preferred_element_type=jnp.float32) + b2_ref[...]


def _seg_sc(h_hbm, src_hbm, dst_hbm, zero_hbm, out_hbm, idx_s, idx_d, rows,
            acc, sem):
    c = jax.lax.axis_index("c")
    s = jax.lax.axis_index("s")
    wid = c * 16 + s
    pltpu.sync_copy(src_hbm.at[wid], idx_s)
    r0 = s * STRIPE
    pltpu.sync_copy(zero_hbm.at[pl.ds(r0, STRIPE)], acc.at[pl.ds(r0, STRIPE)])
    plsc.subcore_barrier()

    def group(j0, carry):
        handles = [pltpu.async_copy(h_hbm.at[idx_s.at[j0 + g]], rows.at[g], sem)
                   for g in range(GDEPTH)]
        pltpu.sync_copy(dst_hbm.at[wid, pl.ds(j0, GDEPTH)], idx_d)
        for g in range(GDEPTH):
            handles[g].wait()
            pltpu.sync_copy(rows.at[g], acc.at[idx_d.at[g]], add=True)
        return carry

    jax.lax.fori_loop(0, NCH // GDEPTH, lambda i, c: group(i * GDEPTH, c), 0)
    plsc.subcore_barrier()
    pltpu.sync_copy(acc.at[pl.ds(r0, STRIPE)], out_hbm.at[c, pl.ds(r0, STRIPE)])


_seg_kernel = functools.partial(
    pl.kernel,
    out_type=jax.ShapeDtypeStruct((2, NPAD, HD), jnp.float32),
    mesh=plsc.VectorSubcoreMesh(core_axis_name="c", subcore_axis_name="s"),
    scratch_types=[
        pltpu.VMEM((NCH, ECHUNK), jnp.int32),
        pltpu.VMEM((GDEPTH, ECHUNK), jnp.int32),
        pltpu.VMEM((GDEPTH, ECHUNK, HD), jnp.float32),
        pltpu.VMEM_SHARED((NPAD, HD), jnp.float32),
        pltpu.SemaphoreType.DMA,
    ],
)(_seg_sc)


def _tile_spec(i=None):
    return pl.BlockSpec((WIN, HD), lambda t: (t, 0))


def _full(shape):
    return pl.BlockSpec(shape, lambda t: tuple(0 for _ in shape))


def _enc(x, w, b):
    return pl.pallas_call(
        _enc_body,
        grid=(NT,),
        in_specs=[_tile_spec(), _full((HD, HD)), _full((1, HD))],
        out_specs=_tile_spec(),
        out_shape=jax.ShapeDtypeStruct((NPAD, HD), jnp.float32),
    )(x, w, b)


def _ffn(h, w1, b1, w2, b2):
    return pl.pallas_call(
        _ffn_body,
        grid=(NT,),
        in_specs=[_tile_spec(), _full((HD, 2 * HD)), _full((1, 2 * HD)),
                  _full((2 * HD, HD)), _full((1, HD))],
        out_specs=[_tile_spec(), _full((8, HD))],
        out_shape=[jax.ShapeDtypeStruct((NPAD, HD), jnp.float32),
                   jax.ShapeDtypeStruct((8, HD), jnp.float32)],
    )(h, w1, b1, w2, b2)


def _bn(h, st, sc, bi):
    return pl.pallas_call(
        _bn_body,
        grid=(NT,),
        in_specs=[_tile_spec(), _full((8, HD)), _full((1, HD)), _full((1, HD))],
        out_specs=_tile_spec(),
        out_shape=jax.ShapeDtypeStruct((NPAD, HD), jnp.float32),
    )(h, st, sc, bi)


def _attn(offs, cnts, h, wq, wk):
    return pl.pallas_call(
        _attn_body,
        grid_spec=pltpu.PrefetchScalarGridSpec(
            num_scalar_prefetch=2,
            grid=(BG,),
            in_specs=[pl.BlockSpec((NPAD, HD), lambda b, o, c: (0, 0)),
                      pl.BlockSpec((2, HD, HD), lambda b, o, c: (0, 0, 0)),
                      pl.BlockSpec((2, HD, HD), lambda b, o, c: (0, 0, 0))],
            out_specs=pl.BlockSpec((1, WIN, HD), lambda b, o, c: (b, 0, 0)),
        ),
        out_shape=jax.ShapeDtypeStruct((BG, WIN, HD), jnp.float32),
    )(offs, cnts, h, wq, wk)


def _gin(offs, cnts, h, parts, eps, agg, w1, b1, w2, b2, with_pool):
    body = functools.partial(_gin_body, with_pool=with_pool)
    out_specs = [pl.BlockSpec((NPAD, HD), lambda b, o, c: (0, 0)),
                 pl.BlockSpec((1, 8, HD), lambda b, o, c: (b, 0, 0))]
    out_shape = [jax.ShapeDtypeStruct((NPAD, HD), jnp.float32),
                 jax.ShapeDtypeStruct((BG, 8, HD), jnp.float32)]
    return pl.pallas_call(
        body,
        grid_spec=pltpu.PrefetchScalarGridSpec(
            num_scalar_prefetch=2,
            grid=(BG,),
            in_specs=[pl.BlockSpec((NPAD, HD), lambda b, o, c: (0, 0)),
                      pl.BlockSpec((2, NPAD, HD), lambda b, o, c: (0, 0, 0)),
                      pl.BlockSpec((1, 1), lambda b, o, c: (0, 0)),
                      pl.BlockSpec((1, WIN, HD), lambda b, o, c: (b, 0, 0)),
                      pl.BlockSpec((HD, HD), lambda b, o, c: (0, 0)),
                      pl.BlockSpec((1, HD), lambda b, o, c: (0, 0)),
                      pl.BlockSpec((HD, HD), lambda b, o, c: (0, 0)),
                      pl.BlockSpec((1, HD), lambda b, o, c: (0, 0))],
            out_specs=out_specs,
        ),
        out_shape=out_shape,
    )(offs, cnts, h, parts, eps, agg, w1, b1, w2, b2)


def _head(pool, w1, b1, w2, b2):
    return pl.pallas_call(
        _head_body,
        grid=(1,),
        in_specs=[_full((BG, 8, HD)), _full((HD, HD)), _full((1, HD)),
                  _full((HD, HD)), _full((1, HD))],
        out_specs=_full((BG, HD)),
        out_shape=jax.ShapeDtypeStruct((BG, HD), jnp.float32),
    )(pool, w1, b1, w2, b2)


def kernel(x, edge_index, batch, enc_W, enc_b, tf_W1, tf_b1, tf_W2, tf_b2,
           bn_scale, bn_bias, attn_Wq, attn_Wk, gin_eps, gin_W1, gin_b1,
           gin_W2, gin_b2, mlp_W1, mlp_b1, mlp_W2, mlp_b2):
    f32 = jnp.float32
    counts = jnp.bincount(batch, length=BG).astype(jnp.int32)
    offs = jnp.concatenate([jnp.zeros((1,), jnp.int32),
                            jnp.cumsum(counts)[:-1].astype(jnp.int32)])

    xp = jnp.zeros((NPAD, HD), f32).at[:NN].set(x)

    tot = NWORK * NCH * ECHUNK
    src = jnp.zeros((tot,), jnp.int32).at[:NE].set(edge_index[0])
    dst = jnp.full((tot,), NN, jnp.int32).at[:NE].set(edge_index[1])
    srcr = src.reshape(NWORK, NCH, ECHUNK)
    dstr = dst.reshape(NWORK, NCH, ECHUNK)
    zeros_rows = jnp.zeros((NPAD, HD), f32)

    h = _enc(xp, enc_W, enc_b.reshape(1, HD))
    for l in range(2):
        h1, st = _ffn(h, tf_W1[l], tf_b1[l].reshape(1, -1), tf_W2[l],
                      tf_b2[l].reshape(1, HD))
        h2 = _bn(h1, st, bn_scale[l].reshape(1, HD), bn_bias[l].reshape(1, HD))
        agg = _attn(offs, counts, h2, attn_Wq[l], attn_Wk[l])
        parts = _seg_kernel(h2, srcr, dstr, zeros_rows)
        h, pool = _gin(offs, counts, h2, parts, gin_eps[l].reshape(1, 1),
                       agg, gin_W1[l], gin_b1[l].reshape(1, HD), gin_W2[l],
                       gin_b2[l].reshape(1, HD), with_pool=(l == 1))

    w2p = jnp.zeros((HD, HD), f32).at[:, :mlp_W2.shape[1]].set(mlp_W2)
    b2p = jnp.zeros((1, HD), f32).at[0, :mlp_b2.shape[0]].set(mlp_b2)
    out = _head(pool, mlp_W1, mlp_b1.reshape(1, HD), w2p, b2p)
    return out[:, :mlp_W2.shape[1]]


# speedup vs baseline: 1.1874x; 1.1874x over previous
import functools
import math

import jax
import jax.numpy as jnp
from jax.experimental import pallas as pl
from jax.experimental.pallas import tpu as pltpu
from jax.experimental.pallas import tpu_sc as plsc

NN = 10000
BG = 100
WIN = 256
HD = 128
KTOP = 16
NE = 320000
NPAD = 10496
NT = NPAD // WIN
NWORK = 32
ECHUNK = 128
NCH = 79
STRIPE = NPAD // 16


def _enc_body(x_ref, w_ref, b_ref, o_ref):
    o_ref[...] = jnp.dot(x_ref[...], w_ref[...],
                         preferred_element_type=jnp.float32) + b_ref[...]


def _ffn_body(h_ref, w1_ref, b1_ref, w2_ref, b2_ref, o_ref, st_ref):
    t = pl.program_id(0)
    h = h_ref[...]
    ff = jnp.maximum(jnp.dot(h, w1_ref[...], preferred_element_type=jnp.float32)
                     + b1_ref[...], 0.0)
    o = h + (jnp.dot(ff, w2_ref[...], preferred_element_type=jnp.float32)
             + b2_ref[...])
    o_ref[...] = o
    rows = t * WIN + jax.lax.broadcasted_iota(jnp.int32, (WIN, 1), 0)
    om = jnp.where(rows < NN, o, 0.0)
    s1 = jnp.sum(om, axis=0)
    s2 = jnp.sum(om * om, axis=0)
    blk = jnp.concatenate([s1[None, :], s2[None, :],
                           jnp.zeros((6, HD), jnp.float32)], axis=0)

    @pl.when(t == 0)
    def _():
        st_ref[...] = blk

    @pl.when(t > 0)
    def _():
        st_ref[...] = st_ref[...] + blk


def _bn_body(h_ref, st_ref, sc_ref, bi_ref, o_ref):
    mu = st_ref[0, :] / float(NN)
    var = st_ref[1, :] / float(NN) - mu * mu
    inv = jax.lax.rsqrt(var + 1e-5)
    o_ref[...] = sc_ref[...] * (h_ref[...] - mu) * inv + bi_ref[...]


def _attn_body(offs_ref, cnts_ref, h_ref, wq_ref, wk_ref, o_ref):
    b = pl.program_id(0)
    off = offs_ref[b]
    cnt = cnts_ref[b]
    win = h_ref[pl.ds(off, WIN), :]
    rows = jax.lax.broadcasted_iota(jnp.int32, (WIN, 1), 0)
    winz = jnp.where(rows < cnt, win, 0.0)
    q0 = jnp.dot(winz, wq_ref[0], preferred_element_type=jnp.float32)
    k0 = jnp.dot(winz, wk_ref[0], preferred_element_type=jnp.float32)
    q1 = jnp.dot(winz, wq_ref[1], preferred_element_type=jnp.float32)
    k1 = jnp.dot(winz, wk_ref[1], preferred_element_type=jnp.float32)
    dn = (([1], [1]), ((), ()))
    sq = jnp.float32(math.sqrt(float(HD)))
    s0 = jax.lax.dot_general(q0, k0, dn, preferred_element_type=jnp.float32)
    s1 = jax.lax.dot_general(q1, k1, dn, preferred_element_type=jnp.float32)
    s = (s0 / sq + s1 / sq) * 0.5
    cols = jax.lax.broadcasted_iota(jnp.int32, (WIN, WIN), 1)
    sm = jnp.where(cols < cnt, s, -1e9)

    def body(_, sw):
        m = jnp.max(sw, axis=1, keepdims=True)
        return jnp.where(sw == m, -jnp.inf, sw)

    sw = jax.lax.fori_loop(0, KTOP - 1, body, sm)
    thr = jnp.max(sw, axis=1, keepdims=True)
    wmat = (sm >= thr).astype(jnp.float32)
    o_ref[0] = jnp.dot(wmat, winz, preferred_element_type=jnp.float32,
                       precision=jax.lax.Precision.HIGHEST)


def _gin_body(offs_ref, cnts_ref, h_ref, p_ref, eps_ref, agg_ref, w1_ref,
              b1_ref, w2_ref, b2_ref, o_ref, pool_ref, *, with_pool):
    b = pl.program_id(0)
    off = offs_ref[b]
    cnt = cnts_ref[b]

    @pl.when(b == 0)
    def _():
        o_ref[...] = jnp.zeros((NPAD, HD), jnp.float32)

    hw_in = h_ref[pl.ds(off, WIN), :]
    g = (((1.0 + eps_ref[0, 0]) * hw_in + agg_ref[0])
         + (p_ref[0, pl.ds(off, WIN), :] + p_ref[1, pl.ds(off, WIN), :]))
    hh = jnp.maximum(jnp.dot(g, w1_ref[...], preferred_element_type=jnp.float32)
                     + b1_ref[...], 0.0)
    hw = jnp.maximum(jnp.dot(hh, w2_ref[...], preferred_element_type=jnp.float32)
                     + b2_ref[...], 0.0)
    o_ref[pl.ds(off, WIN), :] = hw
    if with_pool:
        rows = jax.lax.broadcasted_iota(jnp.int32, (WIN, 1), 0)
        hm = jnp.where(rows < cnt, hw, 0.0)
        srow = jnp.sum(hm, axis=0) / jnp.maximum(cnt, 1).astype(jnp.float32)
        pool_ref[0] = jnp.broadcast_to(srow[None, :], (8, HD))


def _head_body(p_ref, w1_ref, b1_ref, w2_ref, b2_ref, o_ref):
    p = p_ref[:, 0, :]
    hh = jnp.maximum(jnp.dot(p, w1_ref[...], preferred_element_type=jnp.float32)
                     + b1_ref[...], 0.0)
    o_ref[...] = jnp.dot(hh, w2_ref[...], preferred_element_type=jnp.float32) + b2_ref[...]


def _seg_sc(h_hbm, src_hbm, dst_hbm, zero_hbm, out_hbm, idx_s, idx_d, rows,
            acc, sem):
    c = jax.lax.axis_index("c")
    s = jax.lax.axis_index("s")
    wid = c * 16 + s
    pltpu.sync_copy(src_hbm.at[wid], idx_s)
    pltpu.sync_copy(dst_hbm.at[wid], idx_d)
    r0 = s * STRIPE
    pltpu.sync_copy(zero_hbm.at[pl.ds(r0, STRIPE)], acc.at[pl.ds(r0, STRIPE)])
    plsc.subcore_barrier()

    def chunk(j, carry):
        pltpu.async_copy(h_hbm.at[idx_s.at[j]], rows, sem).wait()
        pltpu.sync_copy(rows, acc.at[idx_d.at[j]], add=True)
        return carry

    jax.lax.fori_loop(0, NCH, chunk, 0)
    plsc.subcore_barrier()
    pltpu.sync_copy(acc.at[pl.ds(r0, STRIPE)], out_hbm.at[c, pl.ds(r0, STRIPE)])


_seg_kernel = functools.partial(
    pl.kernel,
    out_type=jax.ShapeDtypeStruct((2, NPAD, HD), jnp.float32),
    mesh=plsc.VectorSubcoreMesh(core_axis_name="c", subcore_axis_name="s"),
    scratch_types=[
        pltpu.VMEM((NCH, ECHUNK), jnp.int32),
        pltpu.VMEM((NCH, ECHUNK), jnp.int32),
        pltpu.VMEM((ECHUNK, HD), jnp.float32),
        pltpu.VMEM_SHARED((NPAD, HD), jnp.float32),
        pltpu.SemaphoreType.DMA,
    ],
)(_seg_sc)


def _tile_spec(i=None):
    return pl.BlockSpec((WIN, HD), lambda t: (t, 0))


def _full(shape):
    return pl.BlockSpec(shape, lambda t: tuple(0 for _ in shape))


def _enc(x, w, b):
    return pl.pallas_call(
        _enc_body,
        grid=(NT,),
        in_specs=[_tile_spec(), _full((HD, HD)), _full((1, HD))],
        out_specs=_tile_spec(),
        out_shape=jax.ShapeDtypeStruct((NPAD, HD), jnp.float32),
    )(x, w, b)


def _ffn(h, w1, b1, w2, b2):
    return pl.pallas_call(
        _ffn_body,
        grid=(NT,),
        in_specs=[_tile_spec(), _full((HD, 2 * HD)), _full((1, 2 * HD)),
                  _full((2 * HD, HD)), _full((1, HD))],
        out_specs=[_tile_spec(), _full((8, HD))],
        out_shape=[jax.ShapeDtypeStruct((NPAD, HD), jnp.float32),
                   jax.ShapeDtypeStruct((8, HD), jnp.float32)],
    )(h, w1, b1, w2, b2)


def _bn(h, st, sc, bi):
    return pl.pallas_call(
        _bn_body,
        grid=(NT,),
        in_specs=[_tile_spec(), _full((8, HD)), _full((1, HD)), _full((1, HD))],
        out_specs=_tile_spec(),
        out_shape=jax.ShapeDtypeStruct((NPAD, HD), jnp.float32),
    )(h, st, sc, bi)


def _attn(offs, cnts, h, wq, wk):
    return pl.pallas_call(
        _attn_body,
        grid_spec=pltpu.PrefetchScalarGridSpec(
            num_scalar_prefetch=2,
            grid=(BG,),
            in_specs=[pl.BlockSpec((NPAD, HD), lambda b, o, c: (0, 0)),
                      pl.BlockSpec((2, HD, HD), lambda b, o, c: (0, 0, 0)),
                      pl.BlockSpec((2, HD, HD), lambda b, o, c: (0, 0, 0))],
            out_specs=pl.BlockSpec((1, WIN, HD), lambda b, o, c: (b, 0, 0)),
        ),
        out_shape=jax.ShapeDtypeStruct((BG, WIN, HD), jnp.float32),
    )(offs, cnts, h, wq, wk)


def _gin(offs, cnts, h, parts, eps, agg, w1, b1, w2, b2, with_pool):
    body = functools.partial(_gin_body, with_pool=with_pool)
    out_specs = [pl.BlockSpec((NPAD, HD), lambda b, o, c: (0, 0)),
                 pl.BlockSpec((1, 8, HD), lambda b, o, c: (b, 0, 0))]
    out_shape = [jax.ShapeDtypeStruct((NPAD, HD), jnp.float32),
                 jax.ShapeDtypeStruct((BG, 8, HD), jnp.float32)]
    return pl.pallas_call(
        body,
        grid_spec=pltpu.PrefetchScalarGridSpec(
            num_scalar_prefetch=2,
            grid=(BG,),
            in_specs=[pl.BlockSpec((NPAD, HD), lambda b, o, c: (0, 0)),
                      pl.BlockSpec((2, NPAD, HD), lambda b, o, c: (0, 0, 0)),
                      pl.BlockSpec((1, 1), lambda b, o, c: (0, 0)),
                      pl.BlockSpec((1, WIN, HD), lambda b, o, c: (b, 0, 0)),
                      pl.BlockSpec((HD, HD), lambda b, o, c: (0, 0)),
                      pl.BlockSpec((1, HD), lambda b, o, c: (0, 0)),
                      pl.BlockSpec((HD, HD), lambda b, o, c: (0, 0)),
                      pl.BlockSpec((1, HD), lambda b, o, c: (0, 0))],
            out_specs=out_specs,
        ),
        out_shape=out_shape,
    )(offs, cnts, h, parts, eps, agg, w1, b1, w2, b2)


def _head(pool, w1, b1, w2, b2):
    return pl.pallas_call(
        _head_body,
        grid=(1,),
        in_specs=[_full((BG, 8, HD)), _full((HD, HD)), _full((1, HD)),
                  _full((HD, HD)), _full((1, HD))],
        out_specs=_full((BG, HD)),
        out_shape=jax.ShapeDtypeStruct((BG, HD), jnp.float32),
    )(pool, w1, b1, w2, b2)


def kernel(x, edge_index, batch, enc_W, enc_b, tf_W1, tf_b1, tf_W2, tf_b2,
           bn_scale, bn_bias, attn_Wq, attn_Wk, gin_eps, gin_W1, gin_b1,
           gin_W2, gin_b2, mlp_W1, mlp_b1, mlp_W2, mlp_b2):
    f32 = jnp.float32
    counts = jnp.bincount(batch, length=BG).astype(jnp.int32)
    offs = jnp.concatenate([jnp.zeros((1,), jnp.int32),
                            jnp.cumsum(counts)[:-1].astype(jnp.int32)])

    xp = jnp.zeros((NPAD, HD), f32).at[:NN].set(x)

    tot = NWORK * NCH * ECHUNK
    src = jnp.zeros((tot,), jnp.int32).at[:NE].set(edge_index[0])
    dst = jnp.full((tot,), NN, jnp.int32).at[:NE].set(edge_index[1])
    srcr = src.reshape(NWORK, NCH, ECHUNK)
    dstr = dst.reshape(NWORK, NCH, ECHUNK)
    zeros_rows = jnp.zeros((NPAD, HD), f32)

    h = _enc(xp, enc_W, enc_b.reshape(1, HD))
    for l in range(2):
        h1, st = _ffn(h, tf_W1[l], tf_b1[l].reshape(1, -1), tf_W2[l],
                      tf_b2[l].reshape(1, HD))
        h2 = _bn(h1, st, bn_scale[l].reshape(1, HD), bn_bias[l].reshape(1, HD))
        agg = _attn(offs, counts, h2, attn_Wq[l], attn_Wk[l])
        parts = _seg_kernel(h2, srcr, dstr, zeros_rows)
        h, pool = _gin(offs, counts, h2, parts, gin_eps[l].reshape(1, 1),
                       agg, gin_W1[l], gin_b1[l].reshape(1, HD), gin_W2[l],
                       gin_b2[l].reshape(1, HD), with_pool=(l == 1))

    w2p = jnp.zeros((HD, HD), f32).at[:, :mlp_W2.shape[1]].set(mlp_W2)
    b2p = jnp.zeros((1, HD), f32).at[0, :mlp_b2.shape[0]].set(mlp_b2)
    out = _head(pool, mlp_W1, mlp_b1.reshape(1, HD), w2p, b2p)
    return out[:, :mlp_W2.shape[1]]
